# baseline (speedup 1.0000x reference)
import functools

import jax
import jax.numpy as jnp
from jax import lax
from jax.experimental import pallas as pl
from jax.experimental.pallas import tpu as pltpu
from jax.experimental.pallas import tpu_sc as plsc

_BATCH = 4
_ROWS = 4096
_CH = 1024
_NWORKERS = 32
_CHUNK = 32
_CHUNKS_PER_W = _ROWS // (_NWORKERS * _CHUNK)

_mesh = plsc.VectorSubcoreMesh(core_axis_name="c", subcore_axis_name="s")


@functools.partial(
    pl.kernel,
    out_type=jax.ShapeDtypeStruct((_BATCH, _ROWS, _CH), jnp.float32),
    mesh=_mesh,
    scratch_types=[
        pltpu.VMEM((2, _CHUNK, _CH), jnp.float32),
        pltpu.SemaphoreType.DMA,
        pltpu.SemaphoreType.DMA,
    ],
)
def _sc_broadcast(x_hbm, o_hbm, buf, sem_in, sem_out):
    wid = lax.axis_index("s") * 2 + lax.axis_index("c")
    base = wid * _CHUNKS_PER_W * _CHUNK

    def row0(j):
        return base + j * _CHUNK

    n = _CHUNKS_PER_W
    in_copies = [None] * n
    out_copies = [None] * n
    in_copies[0] = pltpu.async_copy(
        x_hbm.at[pl.ds(row0(0), _CHUNK)], buf.at[0], sem_in)
    for j in range(n):
        slot = j % 2
        if j + 1 < n:
            if j - 1 >= 0:
                for c in out_copies[j - 1]:
                    c.wait()
            in_copies[j + 1] = pltpu.async_copy(
                x_hbm.at[pl.ds(row0(j + 1), _CHUNK)], buf.at[1 - slot], sem_in)
        in_copies[j].wait()
        out_copies[j] = [
            pltpu.async_copy(
                buf.at[slot], o_hbm.at[b, pl.ds(row0(j), _CHUNK)], sem_out)
            for b in range(_BATCH)
        ]
    for j in (n - 2, n - 1):
        for c in out_copies[j]:
            c.wait()


def kernel(pos_embs, batch_size, index_dim):
    del batch_size, index_dim
    return _sc_broadcast(pos_embs)
